# baseline (speedup 1.0000x reference)
import functools

import jax
import jax.numpy as jnp
from jax import lax
from jax.experimental import pallas as pl
from jax.experimental.pallas import tpu as pltpu
from jax.experimental.pallas import tpu_sc as plsc


def _matmul_rows_body(x_ref, w_ref, o_ref):
    o_ref[...] = jnp.dot(x_ref[...], w_ref[...], preferred_element_type=jnp.float32)


def _edge_matmul(edge_rep, WbT, block):
    E = edge_rep.shape[0]
    G = WbT.shape[1]
    return pl.pallas_call(
        _matmul_rows_body,
        grid=(E // block,),
        in_specs=[
            pl.BlockSpec((block, edge_rep.shape[1]), lambda i: (i, 0)),
            pl.BlockSpec(WbT.shape, lambda i: (0, 0)),
        ],
        out_specs=pl.BlockSpec((block, G), lambda i: (i, 0)),
        out_shape=jax.ShapeDtypeStruct((E, G), jnp.float32),
    )(edge_rep, WbT)


def _node_table_body(hid, nr_ref, wa_ref, wh_ref, bi_ref, bh_ref, s0_ref, s1_ref,
                     tg_ref, tc_ref):
    s = s0_ref[...] + s1_ref[...]
    h = s[:, :hid]
    tg_ref[...] = (jnp.dot(nr_ref[...], wa_ref[...], preferred_element_type=jnp.float32)
                   + jnp.dot(h, wh_ref[...], preferred_element_type=jnp.float32)
                   + bi_ref[...] + bh_ref[...])
    tc_ref[...] = s


def _node_table(node_rep, WaT, WhhT, b_ih2, b_hh2, S0, S1, hid, block):
    N, REP = node_rep.shape
    G = WaT.shape[1]
    return pl.pallas_call(
        functools.partial(_node_table_body, hid),
        grid=(N // block,),
        in_specs=[
            pl.BlockSpec((block, REP), lambda i: (i, 0)),
            pl.BlockSpec(WaT.shape, lambda i: (0, 0)),
            pl.BlockSpec(WhhT.shape, lambda i: (0, 0)),
            pl.BlockSpec(b_ih2.shape, lambda i: (0, 0)),
            pl.BlockSpec(b_hh2.shape, lambda i: (0, 0)),
            pl.BlockSpec((block, 2 * hid), lambda i: (i, 0)),
            pl.BlockSpec((block, 2 * hid), lambda i: (i, 0)),
        ],
        out_specs=[
            pl.BlockSpec((block, G), lambda i: (i, 0)),
            pl.BlockSpec((block, 2 * hid), lambda i: (i, 0)),
        ],
        out_shape=[
            jax.ShapeDtypeStruct((N, G), jnp.float32),
            jax.ShapeDtypeStruct((N, 2 * hid), jnp.float32),
        ],
    )(node_rep, WaT, WhhT, b_ih2, b_hh2, S0, S1)


def _final_body(hid, nr_ref, s0_ref, s1_ref, wn_ref, wh_ref, b_ref, o_ref):
    h = s0_ref[:, :hid] + s1_ref[:, :hid]
    o_ref[...] = jax.nn.relu(
        jnp.dot(nr_ref[...], wn_ref[...], preferred_element_type=jnp.float32)
        + jnp.dot(h, wh_ref[...], preferred_element_type=jnp.float32)
        + b_ref[...])


def _final_mlp(node_rep, S0, S1, WnT, WhT, b2, hid, block):
    N, REP = node_rep.shape
    return pl.pallas_call(
        functools.partial(_final_body, hid),
        grid=(N // block,),
        in_specs=[
            pl.BlockSpec((block, REP), lambda i: (i, 0)),
            pl.BlockSpec((block, 2 * hid), lambda i: (i, 0)),
            pl.BlockSpec((block, 2 * hid), lambda i: (i, 0)),
            pl.BlockSpec(WnT.shape, lambda i: (0, 0)),
            pl.BlockSpec(WhT.shape, lambda i: (0, 0)),
            pl.BlockSpec(b2.shape, lambda i: (0, 0)),
        ],
        out_specs=pl.BlockSpec((block, REP), lambda i: (i, 0)),
        out_shape=jax.ShapeDtypeStruct((N, REP), jnp.float32),
    )(node_rep, S0, S1, WnT, WhT, b2)


def _sigmoid(x):
    return 1.0 / (1.0 + jnp.exp(-x))


def _tanh(x):
    return 2.0 / (1.0 + jnp.exp(-2.0 * x)) - 1.0


def _make_edge_sweep(N, E, hid):
    info = plsc.get_sparse_core_info()
    NC, NS, L = info.num_cores, info.num_subcores, info.num_lanes
    BE = 32
    t1 = (E // (NC * NS)) // BE
    t0 = (E - NS * t1 * BE) // (NS * BE)
    assert NS * (t0 + t1) * BE == E, (t0, t1)
    E0 = NS * t0 * BE
    R0Z = (N // NS) // 8 * 8
    TLZ = N - NS * R0Z
    NZC = R0Z // BE
    ZTL = R0Z - NZC * BE
    G = 4 * hid
    H2 = 2 * hid
    NV = hid // L
    assert TLZ % 8 == 0 and TLZ <= BE and ZTL % 8 == 0

    mesh = plsc.VectorSubcoreMesh(core_axis_name="c", subcore_axis_name="s")

    @functools.partial(
        pl.kernel,
        out_type=jax.ShapeDtypeStruct((NC, N, H2), jnp.float32),
        mesh=mesh,
        scratch_types=[
            pltpu.VMEM((BE,), jnp.int32),
            pltpu.VMEM((BE,), jnp.int32),
            pltpu.VMEM((BE,), jnp.int32),
            pltpu.VMEM((BE,), jnp.int32),
            pltpu.VMEM((BE, G), jnp.float32),
            pltpu.VMEM((BE, G), jnp.float32),
            pltpu.VMEM((BE, G), jnp.float32),
            pltpu.VMEM((BE, G), jnp.float32),
            pltpu.VMEM((BE, H2), jnp.float32),
            pltpu.VMEM((BE, H2), jnp.float32),
            pltpu.VMEM((BE, H2), jnp.float32),
            pltpu.VMEM_SHARED((N, H2), jnp.float32),
            pltpu.SemaphoreType.DMA,
            pltpu.SemaphoreType.DMA,
        ],
    )
    def sweep(tg_hbm, tc_hbm, b_hbm, src_hbm, dst_hbm, out_hbm,
              srcv0, srcv1, dstv0, dstv1, bv0, bv1, gv0, gv1, cv0, cv1,
              mv, acc_sh, sem0, sem1):
        cid = lax.axis_index("c")
        sid = lax.axis_index("s")
        srcv = (srcv0, srcv1)
        dstv = (dstv0, dstv1)
        bv = (bv0, bv1)
        gv = (gv0, gv1)
        cv = (cv0, cv1)
        sem = (sem0, sem1)

        trips = jnp.where(cid == 0, t0, t1)
        ebase = jnp.where(cid == 0, sid * (t0 * BE), E0 + sid * (t1 * BE))

        def _z(r, _):
            for j in range(H2 // L):
                mv[r, pl.ds(j * L, L)] = jnp.zeros((L,), jnp.float32)
            return 0
        lax.fori_loop(0, BE, _z, 0)
        for k in range(NZC):
            pltpu.sync_copy(mv, acc_sh.at[pl.ds(sid * R0Z + k * BE, BE)])
        if ZTL:
            pltpu.sync_copy(mv.at[pl.ds(0, ZTL)],
                            acc_sh.at[pl.ds(sid * R0Z + NZC * BE, ZTL)])

        @pl.when(sid == NS - 1)
        def _zero_tail():
            pltpu.sync_copy(mv.at[pl.ds(0, TLZ)], acc_sh.at[pl.ds(NS * R0Z, TLZ)])

        plsc.subcore_barrier()

        def copy_args(p, blk):
            off = pl.multiple_of(ebase + jnp.minimum(blk, trips - 1) * BE, 8)
            return ((b_hbm.at[pl.ds(off, BE)], bv[p], sem[p]),
                    (tg_hbm.at[srcv[p]], gv[p], sem[p]),
                    (tc_hbm.at[srcv[p]], cv[p], sem[p]))

        def load_and_fire(p, blk):
            off = pl.multiple_of(ebase + jnp.minimum(blk, trips - 1) * BE, 8)
            pltpu.sync_copy(src_hbm.at[pl.ds(off, BE)], srcv[p])
            pltpu.sync_copy(dst_hbm.at[pl.ds(off, BE)], dstv[p])
            for a in copy_args(p, blk):
                pltpu.async_copy(*a)

        def wait_args(p, blk):
            for a in copy_args(p, blk):
                pltpu.make_async_copy(*a).wait()

        load_and_fire(0, 0)
        load_and_fire(1, 1)

        def half_step(p, blk):
            wait_args(p, blk)

            @plsc.parallel_loop(0, BE, step=1, unroll=2)
            def edge_body(e):
                ex = []
                for v in range(NV):
                    o0 = v * L
                    xi = bv[p][e, pl.ds(o0, L)] + gv[p][e, pl.ds(o0, L)]
                    xf = bv[p][e, pl.ds(hid + o0, L)] + gv[p][e, pl.ds(hid + o0, L)]
                    xg = bv[p][e, pl.ds(2 * hid + o0, L)] + gv[p][e, pl.ds(2 * hid + o0, L)]
                    xo = bv[p][e, pl.ds(3 * hid + o0, L)] + gv[p][e, pl.ds(3 * hid + o0, L)]
                    ei = jnp.exp(-xi)
                    ef = jnp.exp(-xf)
                    eg = jnp.exp(-2.0 * jnp.maximum(xg, -30.0))
                    eo = jnp.exp(-xo)
                    ex.append((ei, ef, eg, eo))
                mid = []
                for v in range(NV):
                    ei, ef, eg, eo = ex[v]
                    cc = cv[p][e, pl.ds(hid + v * L, L)]
                    rf = cc / (1.0 + ef)
                    rig = (1.0 - eg) / ((1.0 + ei) * (1.0 + eg))
                    c2 = rf + rig
                    mid.append((c2, 1.0 + eo))
                et = [jnp.exp(-2.0 * jnp.maximum(mid[v][0], -30.0)) for v in range(NV)]
                for v in range(NV):
                    c2, ao = mid[v]
                    t = et[v]
                    h2 = (1.0 - t) / ((1.0 + t) * ao)
                    mv[e, pl.ds(v * L, L)] = h2
                    mv[e, pl.ds(hid + v * L, L)] = c2
            pltpu.sync_copy(mv, acc_sh.at[dstv[p]], add=True)
            load_and_fire(p, blk + 2)

        def dst_load(p, blk):
            off = pl.multiple_of(ebase + blk * BE, 8)
            pltpu.sync_copy(dst_hbm.at[pl.ds(off, BE)], dstv[p])

        def outer_body(i, _):
            dst_load(0, 2 * i)
            half_step(0, 2 * i)
            dst_load(1, 2 * i + 1)
            half_step(1, 2 * i + 1)
            return 0

        lax.fori_loop(0, trips // 2, outer_body, 0)

        @pl.when(trips % 2 == 1)
        def _odd_tail():
            dst_load(0, trips - 1)
            half_step(0, trips - 1)

        for p in (0, 1):
            wait_args(p, 0)
        plsc.subcore_barrier()
        pltpu.sync_copy(acc_sh.at[pl.ds(sid * R0Z, R0Z)],
                        out_hbm.at[cid, pl.ds(sid * R0Z, R0Z)])

        @pl.when(sid == NS - 1)
        def _dump_tail():
            pltpu.sync_copy(acc_sh.at[pl.ds(NS * R0Z, TLZ)],
                            out_hbm.at[cid, pl.ds(NS * R0Z, TLZ)])

    return sweep


def kernel(node_rep, edge_rep, init_state, W_ih, W_hh, b_ih, b_hh, W_upd, b_upd, edge_index):
    N, REP = node_rep.shape
    E = edge_rep.shape[0]
    hid = W_hh.shape[1]
    hops = 2

    src = edge_index[0]
    dst = edge_index[1]
    WaT = W_ih[:, :REP].T
    WbT = W_ih[:, REP:].T
    WhhT = W_hh.T
    b_ih2 = b_ih.reshape(1, -1)
    b_hh2 = b_hh.reshape(1, -1)
    WnT = W_upd[:, :REP].T
    WhT = W_upd[:, REP:].T
    bu2 = b_upd.reshape(1, -1)

    B = _edge_matmul(edge_rep, WbT, block=2000)
    sweep = _make_edge_sweep(N, E, hid)

    S0 = init_state.reshape(N, 2 * hid)
    S1 = jnp.zeros_like(S0)
    for _ in range(hops):
        Tg, Tc = _node_table(node_rep, WaT, WhhT, b_ih2, b_hh2, S0, S1, hid, block=2000)
        acc = sweep(Tg, Tc, B, src, dst)
        S0 = acc[0]
        S1 = acc[1]

    return _final_mlp(node_rep, S0, S1, WnT, WhT, bu2, hid, block=2000)


# speedup vs baseline: 1.1548x; 1.1548x over previous
import functools

import jax
import jax.numpy as jnp
from jax import lax
from jax.experimental import pallas as pl
from jax.experimental.pallas import tpu as pltpu
from jax.experimental.pallas import tpu_sc as plsc


def _matmul_rows_body(x_ref, w_ref, o_ref):
    o_ref[...] = jnp.dot(x_ref[...], w_ref[...], preferred_element_type=jnp.float32)


def _edge_matmul(edge_rep, WbT, block):
    E = edge_rep.shape[0]
    G = WbT.shape[1]
    return pl.pallas_call(
        _matmul_rows_body,
        grid=(E // block,),
        in_specs=[
            pl.BlockSpec((block, edge_rep.shape[1]), lambda i: (i, 0)),
            pl.BlockSpec(WbT.shape, lambda i: (0, 0)),
        ],
        out_specs=pl.BlockSpec((block, G), lambda i: (i, 0)),
        out_shape=jax.ShapeDtypeStruct((E, G), jnp.float32),
    )(edge_rep, WbT)


def _node_table_body(hid, nr_ref, wa_ref, wh_ref, bi_ref, bh_ref, s0_ref, s1_ref,
                     tg_ref, tc_ref):
    s = s0_ref[...] + s1_ref[...]
    h = s[:, :hid]
    tg_ref[...] = (jnp.dot(nr_ref[...], wa_ref[...], preferred_element_type=jnp.float32)
                   + jnp.dot(h, wh_ref[...], preferred_element_type=jnp.float32)
                   + bi_ref[...] + bh_ref[...])
    tc_ref[...] = s


def _node_table(node_rep, WaT, WhhT, b_ih2, b_hh2, S0, S1, hid, block):
    N, REP = node_rep.shape
    G = WaT.shape[1]
    return pl.pallas_call(
        functools.partial(_node_table_body, hid),
        grid=(N // block,),
        in_specs=[
            pl.BlockSpec((block, REP), lambda i: (i, 0)),
            pl.BlockSpec(WaT.shape, lambda i: (0, 0)),
            pl.BlockSpec(WhhT.shape, lambda i: (0, 0)),
            pl.BlockSpec(b_ih2.shape, lambda i: (0, 0)),
            pl.BlockSpec(b_hh2.shape, lambda i: (0, 0)),
            pl.BlockSpec((block, 2 * hid), lambda i: (i, 0)),
            pl.BlockSpec((block, 2 * hid), lambda i: (i, 0)),
        ],
        out_specs=[
            pl.BlockSpec((block, G), lambda i: (i, 0)),
            pl.BlockSpec((block, 2 * hid), lambda i: (i, 0)),
        ],
        out_shape=[
            jax.ShapeDtypeStruct((N, G), jnp.float32),
            jax.ShapeDtypeStruct((N, 2 * hid), jnp.float32),
        ],
    )(node_rep, WaT, WhhT, b_ih2, b_hh2, S0, S1)


def _final_body(hid, nr_ref, s0_ref, s1_ref, wn_ref, wh_ref, b_ref, o_ref):
    h = s0_ref[:, :hid] + s1_ref[:, :hid]
    o_ref[...] = jax.nn.relu(
        jnp.dot(nr_ref[...], wn_ref[...], preferred_element_type=jnp.float32)
        + jnp.dot(h, wh_ref[...], preferred_element_type=jnp.float32)
        + b_ref[...])


def _final_mlp(node_rep, S0, S1, WnT, WhT, b2, hid, block):
    N, REP = node_rep.shape
    return pl.pallas_call(
        functools.partial(_final_body, hid),
        grid=(N // block,),
        in_specs=[
            pl.BlockSpec((block, REP), lambda i: (i, 0)),
            pl.BlockSpec((block, 2 * hid), lambda i: (i, 0)),
            pl.BlockSpec((block, 2 * hid), lambda i: (i, 0)),
            pl.BlockSpec(WnT.shape, lambda i: (0, 0)),
            pl.BlockSpec(WhT.shape, lambda i: (0, 0)),
            pl.BlockSpec(b2.shape, lambda i: (0, 0)),
        ],
        out_specs=pl.BlockSpec((block, REP), lambda i: (i, 0)),
        out_shape=jax.ShapeDtypeStruct((N, REP), jnp.float32),
    )(node_rep, S0, S1, WnT, WhT, b2)


def _sigmoid(x):
    return 1.0 / (1.0 + jnp.exp(-x))


def _tanh(x):
    return 2.0 / (1.0 + jnp.exp(-2.0 * x)) - 1.0


def _make_edge_sweep(N, E, hid):
    info = plsc.get_sparse_core_info()
    NC, NS, L = info.num_cores, info.num_subcores, info.num_lanes
    BE = 32
    t1 = (E // (NC * NS)) // BE
    t0 = (E - NS * t1 * BE) // (NS * BE)
    assert NS * (t0 + t1) * BE == E, (t0, t1)
    E0 = NS * t0 * BE
    R0Z = (N // NS) // 8 * 8
    TLZ = N - NS * R0Z
    NZC = R0Z // BE
    ZTL = R0Z - NZC * BE
    G = 4 * hid
    H2 = 2 * hid
    NV = hid // L
    assert TLZ % 8 == 0 and TLZ <= BE and ZTL % 8 == 0

    mesh = plsc.VectorSubcoreMesh(core_axis_name="c", subcore_axis_name="s")

    @functools.partial(
        pl.kernel,
        out_type=jax.ShapeDtypeStruct((NC, N, H2), jnp.float32),
        mesh=mesh,
        scratch_types=[
            pltpu.VMEM((BE,), jnp.int32),
            pltpu.VMEM((BE,), jnp.int32),
            pltpu.VMEM((BE,), jnp.int32),
            pltpu.VMEM((BE,), jnp.int32),
            pltpu.VMEM((BE, G), jnp.float32),
            pltpu.VMEM((BE, G), jnp.float32),
            pltpu.VMEM((BE, G), jnp.float32),
            pltpu.VMEM((BE, G), jnp.float32),
            pltpu.VMEM((BE, H2), jnp.float32),
            pltpu.VMEM((BE, H2), jnp.float32),
            pltpu.VMEM((BE, H2), jnp.float32),
            pltpu.VMEM_SHARED((N, H2), jnp.float32),
            pltpu.SemaphoreType.DMA,
            pltpu.SemaphoreType.DMA,
        ],
    )
    def sweep(tg_hbm, tc_hbm, b_hbm, src_hbm, dst_hbm, out_hbm,
              srcv0, srcv1, dstv0, dstv1, bv0, bv1, gv0, gv1, cv0, cv1,
              mv, acc_sh, sem0, sem1):
        cid = lax.axis_index("c")
        sid = lax.axis_index("s")
        srcv = (srcv0, srcv1)
        dstv = (dstv0, dstv1)
        bv = (bv0, bv1)
        gv = (gv0, gv1)
        cv = (cv0, cv1)
        sem = (sem0, sem1)

        trips = jnp.where(cid == 0, t0, t1)
        ebase = jnp.where(cid == 0, sid * (t0 * BE), E0 + sid * (t1 * BE))

        def _z(r, _):
            for j in range(H2 // L):
                mv[r, pl.ds(j * L, L)] = jnp.zeros((L,), jnp.float32)
            return 0
        lax.fori_loop(0, BE, _z, 0)
        for k in range(NZC):
            pltpu.sync_copy(mv, acc_sh.at[pl.ds(sid * R0Z + k * BE, BE)])
        if ZTL:
            pltpu.sync_copy(mv.at[pl.ds(0, ZTL)],
                            acc_sh.at[pl.ds(sid * R0Z + NZC * BE, ZTL)])

        @pl.when(sid == NS - 1)
        def _zero_tail():
            pltpu.sync_copy(mv.at[pl.ds(0, TLZ)], acc_sh.at[pl.ds(NS * R0Z, TLZ)])

        plsc.subcore_barrier()

        def copy_args(p, blk):
            off = pl.multiple_of(ebase + jnp.minimum(blk, trips - 1) * BE, 8)
            return ((b_hbm.at[pl.ds(off, BE)], bv[p], sem[p]),
                    (tg_hbm.at[srcv[p]], gv[p], sem[p]),
                    (tc_hbm.at[srcv[p]], cv[p], sem[p]))

        def load_and_fire(p, blk):
            off = pl.multiple_of(ebase + jnp.minimum(blk, trips - 1) * BE, 8)
            pltpu.sync_copy(src_hbm.at[pl.ds(off, BE)], srcv[p])
            for a in copy_args(p, blk):
                pltpu.async_copy(*a)

        def wait_args(p, blk):
            for a in copy_args(p, blk):
                pltpu.make_async_copy(*a).wait()

        load_and_fire(0, 0)
        load_and_fire(1, 1)

        def half_step(p, blk):
            wait_args(p, blk)

            @plsc.parallel_loop(0, BE, step=1, unroll=2)
            def edge_body(e):
                ex = []
                for v in range(NV):
                    o0 = v * L
                    xi = bv[p][e, pl.ds(o0, L)] + gv[p][e, pl.ds(o0, L)]
                    xf = bv[p][e, pl.ds(hid + o0, L)] + gv[p][e, pl.ds(hid + o0, L)]
                    xg = bv[p][e, pl.ds(2 * hid + o0, L)] + gv[p][e, pl.ds(2 * hid + o0, L)]
                    xo = bv[p][e, pl.ds(3 * hid + o0, L)] + gv[p][e, pl.ds(3 * hid + o0, L)]
                    ei = jnp.exp(-xi)
                    ef = jnp.exp(-xf)
                    eg = jnp.exp(-2.0 * jnp.maximum(xg, -30.0))
                    eo = jnp.exp(-xo)
                    ex.append((ei, ef, eg, eo))
                mid = []
                for v in range(NV):
                    ei, ef, eg, eo = ex[v]
                    cc = cv[p][e, pl.ds(hid + v * L, L)]
                    rf = cc / (1.0 + ef)
                    rig = (1.0 - eg) / ((1.0 + ei) * (1.0 + eg))
                    c2 = rf + rig
                    mid.append((c2, 1.0 + eo))
                et = [jnp.exp(-2.0 * jnp.maximum(mid[v][0], -30.0)) for v in range(NV)]
                for v in range(NV):
                    c2, ao = mid[v]
                    t = et[v]
                    h2 = (1.0 - t) / ((1.0 + t) * ao)
                    mv[e, pl.ds(v * L, L)] = h2
                    mv[e, pl.ds(hid + v * L, L)] = c2
            pltpu.sync_copy(mv, acc_sh.at[dstv[p]], add=True)
            load_and_fire(p, blk + 2)

        def dst_load(p, blk):
            off = pl.multiple_of(ebase + blk * BE, 8)
            pltpu.sync_copy(dst_hbm.at[pl.ds(off, BE)], dstv[p])

        def outer_body(i, _):
            dst_load(0, 2 * i)
            half_step(0, 2 * i)
            dst_load(1, 2 * i + 1)
            half_step(1, 2 * i + 1)
            return 0

        lax.fori_loop(0, trips // 2, outer_body, 0)

        @pl.when(trips % 2 == 1)
        def _odd_tail():
            dst_load(0, trips - 1)
            half_step(0, trips - 1)

        for p in (0, 1):
            wait_args(p, 0)
        plsc.subcore_barrier()
        pltpu.sync_copy(acc_sh.at[pl.ds(sid * R0Z, R0Z)],
                        out_hbm.at[cid, pl.ds(sid * R0Z, R0Z)])

        @pl.when(sid == NS - 1)
        def _dump_tail():
            pltpu.sync_copy(acc_sh.at[pl.ds(NS * R0Z, TLZ)],
                            out_hbm.at[cid, pl.ds(NS * R0Z, TLZ)])

    return sweep


def kernel(node_rep, edge_rep, init_state, W_ih, W_hh, b_ih, b_hh, W_upd, b_upd, edge_index):
    N, REP = node_rep.shape
    E = edge_rep.shape[0]
    hid = W_hh.shape[1]
    hops = 2

    src = edge_index[0]
    dst = edge_index[1]
    WaT = W_ih[:, :REP].T
    WbT = W_ih[:, REP:].T
    WhhT = W_hh.T
    b_ih2 = b_ih.reshape(1, -1)
    b_hh2 = b_hh.reshape(1, -1)
    WnT = W_upd[:, :REP].T
    WhT = W_upd[:, REP:].T
    bu2 = b_upd.reshape(1, -1)

    B = _edge_matmul(edge_rep, WbT, block=2000)
    sweep = _make_edge_sweep(N, E, hid)

    S0 = init_state.reshape(N, 2 * hid)
    S1 = jnp.zeros_like(S0)
    for _ in range(hops):
        Tg, Tc = _node_table(node_rep, WaT, WhhT, b_ih2, b_hh2, S0, S1, hid, block=2000)
        acc = sweep(Tg, Tc, B, src, dst)
        S0 = acc[0]
        S1 = acc[1]

    return _final_mlp(node_rep, S0, S1, WnT, WhT, bu2, hid, block=2000)


# speedup vs baseline: 1.1894x; 1.0299x over previous
import functools

import jax
import jax.numpy as jnp
from jax import lax
from jax.experimental import pallas as pl
from jax.experimental.pallas import tpu as pltpu
from jax.experimental.pallas import tpu_sc as plsc


def _matmul_rows_body(x_ref, w_ref, o_ref):
    o_ref[...] = jnp.dot(x_ref[...], w_ref[...], preferred_element_type=jnp.float32)


def _edge_matmul(edge_rep, WbT, block):
    E = edge_rep.shape[0]
    G = WbT.shape[1]
    return pl.pallas_call(
        _matmul_rows_body,
        grid=(E // block,),
        in_specs=[
            pl.BlockSpec((block, edge_rep.shape[1]), lambda i: (i, 0)),
            pl.BlockSpec(WbT.shape, lambda i: (0, 0)),
        ],
        out_specs=pl.BlockSpec((block, G), lambda i: (i, 0)),
        out_shape=jax.ShapeDtypeStruct((E, G), jnp.float32),
    )(edge_rep, WbT)


def _node_table_body(hid, nr_ref, wa_ref, wh_ref, bi_ref, bh_ref, s0_ref, s1_ref,
                     tg_ref, tc_ref):
    s = s0_ref[...] + s1_ref[...]
    h = s[:, :hid]
    tg_ref[...] = (jnp.dot(nr_ref[...], wa_ref[...], preferred_element_type=jnp.float32)
                   + jnp.dot(h, wh_ref[...], preferred_element_type=jnp.float32)
                   + bi_ref[...] + bh_ref[...])
    tc_ref[...] = s


def _node_table(node_rep, WaT, WhhT, b_ih2, b_hh2, S0, S1, hid, block):
    N, REP = node_rep.shape
    G = WaT.shape[1]
    return pl.pallas_call(
        functools.partial(_node_table_body, hid),
        grid=(N // block,),
        in_specs=[
            pl.BlockSpec((block, REP), lambda i: (i, 0)),
            pl.BlockSpec(WaT.shape, lambda i: (0, 0)),
            pl.BlockSpec(WhhT.shape, lambda i: (0, 0)),
            pl.BlockSpec(b_ih2.shape, lambda i: (0, 0)),
            pl.BlockSpec(b_hh2.shape, lambda i: (0, 0)),
            pl.BlockSpec((block, 2 * hid), lambda i: (i, 0)),
            pl.BlockSpec((block, 2 * hid), lambda i: (i, 0)),
        ],
        out_specs=[
            pl.BlockSpec((block, G), lambda i: (i, 0)),
            pl.BlockSpec((block, 2 * hid), lambda i: (i, 0)),
        ],
        out_shape=[
            jax.ShapeDtypeStruct((N, G), jnp.float32),
            jax.ShapeDtypeStruct((N, 2 * hid), jnp.float32),
        ],
    )(node_rep, WaT, WhhT, b_ih2, b_hh2, S0, S1)


def _final_body(hid, nr_ref, s0_ref, s1_ref, wn_ref, wh_ref, b_ref, o_ref):
    h = s0_ref[:, :hid] + s1_ref[:, :hid]
    o_ref[...] = jax.nn.relu(
        jnp.dot(nr_ref[...], wn_ref[...], preferred_element_type=jnp.float32)
        + jnp.dot(h, wh_ref[...], preferred_element_type=jnp.float32)
        + b_ref[...])


def _final_mlp(node_rep, S0, S1, WnT, WhT, b2, hid, block):
    N, REP = node_rep.shape
    return pl.pallas_call(
        functools.partial(_final_body, hid),
        grid=(N // block,),
        in_specs=[
            pl.BlockSpec((block, REP), lambda i: (i, 0)),
            pl.BlockSpec((block, 2 * hid), lambda i: (i, 0)),
            pl.BlockSpec((block, 2 * hid), lambda i: (i, 0)),
            pl.BlockSpec(WnT.shape, lambda i: (0, 0)),
            pl.BlockSpec(WhT.shape, lambda i: (0, 0)),
            pl.BlockSpec(b2.shape, lambda i: (0, 0)),
        ],
        out_specs=pl.BlockSpec((block, REP), lambda i: (i, 0)),
        out_shape=jax.ShapeDtypeStruct((N, REP), jnp.float32),
    )(node_rep, S0, S1, WnT, WhT, b2)


def _sigmoid(x):
    return 1.0 / (1.0 + jnp.exp(-x))


def _tanh(x):
    return 2.0 / (1.0 + jnp.exp(-2.0 * x)) - 1.0


def _make_edge_sweep(N, E, hid):
    info = plsc.get_sparse_core_info()
    NC, NS, L = info.num_cores, info.num_subcores, info.num_lanes
    BE = 32
    t1 = (E // (NC * NS)) // BE
    t0 = (E - NS * t1 * BE) // (NS * BE)
    assert NS * (t0 + t1) * BE == E, (t0, t1)
    E0 = NS * t0 * BE
    R0Z = (N // NS) // 8 * 8
    TLZ = N - NS * R0Z
    NZC = R0Z // BE
    ZTL = R0Z - NZC * BE
    G = 4 * hid
    H2 = 2 * hid
    NV = hid // L
    assert TLZ % 8 == 0 and TLZ <= BE and ZTL % 8 == 0

    mesh = plsc.VectorSubcoreMesh(core_axis_name="c", subcore_axis_name="s")

    @functools.partial(
        pl.kernel,
        out_type=jax.ShapeDtypeStruct((NC, N, H2), jnp.float32),
        mesh=mesh,
        scratch_types=[
            pltpu.VMEM((BE,), jnp.int32),
            pltpu.VMEM((BE,), jnp.int32),
            pltpu.VMEM((BE,), jnp.int32),
            pltpu.VMEM((BE,), jnp.int32),
            pltpu.VMEM((BE, G), jnp.float32),
            pltpu.VMEM((BE, G), jnp.float32),
            pltpu.VMEM((BE, G), jnp.float32),
            pltpu.VMEM((BE, G), jnp.float32),
            pltpu.VMEM((BE, H2), jnp.float32),
            pltpu.VMEM((BE, H2), jnp.float32),
            pltpu.VMEM((BE, H2), jnp.float32),
            pltpu.VMEM_SHARED((N, H2), jnp.float32),
            pltpu.SemaphoreType.DMA,
            pltpu.SemaphoreType.DMA,
            pltpu.SemaphoreType.DMA,
            pltpu.SemaphoreType.DMA,
        ],
    )
    def sweep(tg_hbm, tc_hbm, b_hbm, src_hbm, dst_hbm, out_hbm,
              srcv0, srcv1, dstv0, dstv1, bv0, bv1, gv0, gv1, cv0, cv1,
              mv, acc_sh, sem0, sem1, semd0, semd1):
        cid = lax.axis_index("c")
        sid = lax.axis_index("s")
        srcv = (srcv0, srcv1)
        dstv = (dstv0, dstv1)
        bv = (bv0, bv1)
        gv = (gv0, gv1)
        cv = (cv0, cv1)
        sem = (sem0, sem1)
        semd = (semd0, semd1)

        trips = jnp.where(cid == 0, t0, t1)
        ebase = jnp.where(cid == 0, sid * (t0 * BE), E0 + sid * (t1 * BE))

        def _z(r, _):
            for j in range(H2 // L):
                mv[r, pl.ds(j * L, L)] = jnp.zeros((L,), jnp.float32)
            return 0
        lax.fori_loop(0, BE, _z, 0)
        for k in range(NZC):
            pltpu.sync_copy(mv, acc_sh.at[pl.ds(sid * R0Z + k * BE, BE)])
        if ZTL:
            pltpu.sync_copy(mv.at[pl.ds(0, ZTL)],
                            acc_sh.at[pl.ds(sid * R0Z + NZC * BE, ZTL)])

        @pl.when(sid == NS - 1)
        def _zero_tail():
            pltpu.sync_copy(mv.at[pl.ds(0, TLZ)], acc_sh.at[pl.ds(NS * R0Z, TLZ)])

        plsc.subcore_barrier()

        def copy_args(p, blk):
            off = pl.multiple_of(ebase + jnp.minimum(blk, trips - 1) * BE, 8)
            return ((b_hbm.at[pl.ds(off, BE)], bv[p], sem[p]),
                    (tg_hbm.at[srcv[p]], gv[p], sem[p]),
                    (tc_hbm.at[srcv[p]], cv[p], sem[p]))

        def load_and_fire(p, blk):
            off = pl.multiple_of(ebase + jnp.minimum(blk, trips - 1) * BE, 8)
            pltpu.sync_copy(src_hbm.at[pl.ds(off, BE)], srcv[p])
            for a in copy_args(p, blk):
                pltpu.async_copy(*a)

        def wait_args(p, blk):
            for a in copy_args(p, blk):
                pltpu.make_async_copy(*a).wait()

        load_and_fire(0, 0)
        load_and_fire(1, 1)

        def half_step(p, blk):
            wait_args(p, blk)

            @plsc.parallel_loop(0, BE, step=1, unroll=2)
            def edge_body(e):
                ex = []
                for v in range(NV):
                    o0 = v * L
                    xi = bv[p][e, pl.ds(o0, L)] + gv[p][e, pl.ds(o0, L)]
                    xf = bv[p][e, pl.ds(hid + o0, L)] + gv[p][e, pl.ds(hid + o0, L)]
                    xg = bv[p][e, pl.ds(2 * hid + o0, L)] + gv[p][e, pl.ds(2 * hid + o0, L)]
                    xo = bv[p][e, pl.ds(3 * hid + o0, L)] + gv[p][e, pl.ds(3 * hid + o0, L)]
                    ei = jnp.exp(-xi)
                    ef = jnp.exp(-xf)
                    eg = jnp.exp(-2.0 * jnp.maximum(xg, -30.0))
                    eo = jnp.exp(-xo)
                    ex.append((ei, ef, eg, eo))
                mid = []
                for v in range(NV):
                    ei, ef, eg, eo = ex[v]
                    cc = cv[p][e, pl.ds(hid + v * L, L)]
                    rf = cc / (1.0 + ef)
                    rig = (1.0 - eg) / ((1.0 + ei) * (1.0 + eg))
                    c2 = rf + rig
                    mid.append((c2, 1.0 + eo))
                et = [jnp.exp(-2.0 * jnp.maximum(mid[v][0], -30.0)) for v in range(NV)]
                for v in range(NV):
                    c2, ao = mid[v]
                    t = et[v]
                    h2 = (1.0 - t) / ((1.0 + t) * ao)
                    mv[e, pl.ds(v * L, L)] = h2
                    mv[e, pl.ds(hid + v * L, L)] = c2
            pltpu.sync_copy(mv, acc_sh.at[dstv[p]], add=True)
            load_and_fire(p, blk + 2)

        def dst_load(p, blk):
            off = pl.multiple_of(ebase + blk * BE, 8)
            pltpu.sync_copy(dst_hbm.at[pl.ds(off, BE)], dstv[p])

        def outer_body(i, _):
            dst_load(0, 2 * i)
            half_step(0, 2 * i)
            dst_load(1, 2 * i + 1)
            half_step(1, 2 * i + 1)
            return 0

        lax.fori_loop(0, trips // 2, outer_body, 0)

        @pl.when(trips % 2 == 1)
        def _odd_tail():
            dst_load(0, trips - 1)
            half_step(0, trips - 1)

        for p in (0, 1):
            wait_args(p, 0)
        plsc.subcore_barrier()
        pltpu.sync_copy(acc_sh.at[pl.ds(sid * R0Z, R0Z)],
                        out_hbm.at[cid, pl.ds(sid * R0Z, R0Z)])

        @pl.when(sid == NS - 1)
        def _dump_tail():
            pltpu.sync_copy(acc_sh.at[pl.ds(NS * R0Z, TLZ)],
                            out_hbm.at[cid, pl.ds(NS * R0Z, TLZ)])

    return sweep


def kernel(node_rep, edge_rep, init_state, W_ih, W_hh, b_ih, b_hh, W_upd, b_upd, edge_index):
    N, REP = node_rep.shape
    E = edge_rep.shape[0]
    hid = W_hh.shape[1]
    hops = 2

    src = edge_index[0]
    dst = edge_index[1]
    WaT = W_ih[:, :REP].T
    WbT = W_ih[:, REP:].T
    WhhT = W_hh.T
    b_ih2 = b_ih.reshape(1, -1)
    b_hh2 = b_hh.reshape(1, -1)
    WnT = W_upd[:, :REP].T
    WhT = W_upd[:, REP:].T
    bu2 = b_upd.reshape(1, -1)

    B = _edge_matmul(edge_rep, WbT, block=4000)
    sweep = _make_edge_sweep(N, E, hid)

    S0 = init_state.reshape(N, 2 * hid)
    S1 = jnp.zeros_like(S0)
    for _ in range(hops):
        Tg, Tc = _node_table(node_rep, WaT, WhhT, b_ih2, b_hh2, S0, S1, hid, block=2000)
        acc = sweep(Tg, Tc, B, src, dst)
        S0 = acc[0]
        S1 = acc[1]

    return _final_mlp(node_rep, S0, S1, WnT, WhT, bu2, hid, block=2000)
